# baseline (speedup 1.0000x reference)
import functools

import jax
import jax.numpy as jnp
import numpy as np
from jax import lax
from jax.experimental import pallas as pl
from jax.experimental.pallas import tpu as pltpu
from jax.experimental.pallas import tpu_sc as plsc

D = 300
DP = 304
B = 4096 * 200
NC, NS = 2, 16
NW = NC * NS
BPW = B // NW
C = 64
NCHUNK = BPW // C
NPAIR = NCHUNK // 2
NG = C // 4
NJ = (4 * D) // 16

_W = np.arange(4 * D, dtype=np.int32)
_PROW = (_W // D).astype(np.int32)
_PCOL = (_W % D).astype(np.int32)

_mesh = plsc.VectorSubcoreMesh(core_axis_name="c", subcore_axis_name="s")


@functools.partial(
    pl.kernel,
    out_type=jax.ShapeDtypeStruct((B * D,), jnp.float32),
    mesh=_mesh,
    compiler_params=pltpu.CompilerParams(
        use_tc_tiling_on_sc=False, needs_layout_passes=False),
    scratch_types=[
        pltpu.VMEM((BPW,), jnp.int32),
        pltpu.VMEM((4 * D,), jnp.int32),
        pltpu.VMEM((4 * D,), jnp.int32),
        pltpu.VMEM((C, DP), jnp.float32),
        pltpu.VMEM((C, DP), jnp.float32),
        pltpu.VMEM((C * D,), jnp.float32),
        pltpu.VMEM((C * D,), jnp.float32),
        pltpu.SemaphoreType.DMA,
        pltpu.SemaphoreType.DMA,
        pltpu.SemaphoreType.DMA,
        pltpu.SemaphoreType.DMA,
    ],
)
def _gather_kernel(idx_hbm, table_hbm, prow_hbm, pcol_hbm, out_hbm,
                   idx_v, prow_v, pcol_v, pad0, pad1, cmp0, cmp1,
                   g0, g1, o0, o1):
    wid = lax.axis_index("s") * NC + lax.axis_index("c")
    base = wid * BPW
    pltpu.sync_copy(idx_hbm.at[pl.ds(base, BPW)], idx_v)
    pltpu.sync_copy(prow_hbm, prow_v)
    pltpu.sync_copy(pcol_hbm, pcol_v)

    def gather(g, rows, sem):
        return pltpu.make_async_copy(
            table_hbm.at[idx_v.at[pl.ds(g * C, C)]], rows, sem)

    def store(g, cmp, sem):
        return pltpu.make_async_copy(
            cmp, out_hbm.at[pl.ds((base + g * C) * D, C * D)], sem)

    def compact(pad, cmp):
        def body(j, _):
            pr = prow_v[pl.ds(16 * j, 16)]
            pc = pcol_v[pl.ds(16 * j, 16)]
            for grp in range(NG):
                v = plsc.load_gather(pad, [pr + 4 * grp, pc])
                cmp[pl.ds(1200 * grp + 16 * j, 16)] = v
            return 0
        lax.fori_loop(0, NJ, body, 0)

    gather(0, pad0, g0).start()
    gather(1, pad1, g1).start()

    gather(0, pad0, g0).wait()
    compact(pad0, cmp0)
    store(0, cmp0, o0).start()
    gather(2, pad0, g0).start()
    gather(1, pad1, g1).wait()
    compact(pad1, cmp1)
    store(1, cmp1, o1).start()
    gather(3, pad1, g1).start()

    def pair(i, _):
        a = 2 * i
        gather(a, pad0, g0).wait()
        store(a - 2, cmp0, o0).wait()
        compact(pad0, cmp0)
        store(a, cmp0, o0).start()
        gather(a + 2, pad0, g0).start()
        gather(a + 1, pad1, g1).wait()
        store(a - 1, cmp1, o1).wait()
        compact(pad1, cmp1)
        store(a + 1, cmp1, o1).start()
        gather(a + 3, pad1, g1).start()
        return 0

    lax.fori_loop(1, NPAIR - 1, pair, 0)

    last = NCHUNK - 2
    gather(last, pad0, g0).wait()
    store(last - 2, cmp0, o0).wait()
    compact(pad0, cmp0)
    store(last, cmp0, o0).start()
    gather(last + 1, pad1, g1).wait()
    store(last - 1, cmp1, o1).wait()
    compact(pad1, cmp1)
    store(last + 1, cmp1, o1).start()
    store(last, cmp0, o0).wait()
    store(last + 1, cmp1, o1).wait()


def kernel(inputs, table):
    idx = inputs.reshape(-1).astype(jnp.int32)
    table_p = jnp.pad(table, ((0, 0), (0, DP - D)))
    out = _gather_kernel(idx, table_p, jnp.asarray(_PROW), jnp.asarray(_PCOL))
    return out.reshape(inputs.shape[0], inputs.shape[1], D)


# speedup vs baseline: 2.2303x; 2.2303x over previous
import functools

import jax
import jax.numpy as jnp
from jax import lax
from jax.experimental import pallas as pl
from jax.experimental.pallas import tpu as pltpu
from jax.experimental.pallas import tpu_sc as plsc

D = 300
DP = 384
B = 4096 * 200
NC, NS = 2, 16
NW = NC * NS
BPW = B // NW
C = 128
NCHUNK = BPW // C
NPAIR = NCHUNK // 2

_mesh = plsc.VectorSubcoreMesh(core_axis_name="c", subcore_axis_name="s")


@functools.partial(
    pl.kernel,
    out_type=jax.ShapeDtypeStruct((B, DP), jnp.float32),
    mesh=_mesh,
    scratch_types=[
        pltpu.VMEM((BPW,), jnp.int32),
        pltpu.VMEM((C, DP), jnp.float32),
        pltpu.VMEM((C, DP), jnp.float32),
        pltpu.SemaphoreType.DMA,
        pltpu.SemaphoreType.DMA,
        pltpu.SemaphoreType.DMA,
        pltpu.SemaphoreType.DMA,
    ],
)
def _gather_kernel(idx_hbm, table_hbm, out_hbm,
                   idx_v, rows0, rows1, g0, g1, o0, o1):
    wid = lax.axis_index("s") * NC + lax.axis_index("c")
    base = wid * BPW
    pltpu.sync_copy(idx_hbm.at[pl.ds(base, BPW)], idx_v)

    def gather(g, rows, sem):
        return pltpu.make_async_copy(
            table_hbm.at[idx_v.at[pl.ds(g * C, C)]], rows, sem)

    def store(g, rows, sem):
        return pltpu.make_async_copy(
            rows, out_hbm.at[pl.ds(base + g * C, C)], sem)

    gather(0, rows0, g0).start()
    gather(1, rows1, g1).start()

    def pair(i, _):
        a = 2 * i
        gather(a, rows0, g0).wait()
        store(a, rows0, o0).start()
        gather(a + 1, rows1, g1).wait()
        store(a + 1, rows1, o1).start()
        store(a, rows0, o0).wait()
        gather(a + 2, rows0, g0).start()
        store(a + 1, rows1, o1).wait()
        gather(a + 3, rows1, g1).start()
        return 0

    lax.fori_loop(0, NPAIR - 1, pair, 0)

    last = NCHUNK - 2
    gather(last, rows0, g0).wait()
    store(last, rows0, o0).start()
    gather(last + 1, rows1, g1).wait()
    store(last + 1, rows1, o1).start()
    store(last, rows0, o0).wait()
    store(last + 1, rows1, o1).wait()


def kernel(inputs, table):
    idx = inputs.reshape(-1).astype(jnp.int32)
    table_p = jnp.pad(table, ((0, 0), (0, DP - D)))
    out = _gather_kernel(idx, table_p)
    return out[:, :D].reshape(inputs.shape[0], inputs.shape[1], D)


# speedup vs baseline: 2.7779x; 1.2455x over previous
import functools

import jax
import jax.numpy as jnp
from jax import lax
from jax.experimental import pallas as pl
from jax.experimental.pallas import tpu as pltpu
from jax.experimental.pallas import tpu_sc as plsc

D = 300
DP = 384
B = 4096 * 200
NC, NS = 2, 16
NW = NC * NS
BPW = B // NW
C = 128
NCHUNK = BPW // C
NPAIR = NCHUNK // 2

_mesh = plsc.VectorSubcoreMesh(core_axis_name="c", subcore_axis_name="s")


@functools.partial(
    pl.kernel,
    out_type=jax.ShapeDtypeStruct((B, DP), jnp.float32),
    mesh=_mesh,
    scratch_types=[
        pltpu.VMEM((BPW,), jnp.int32),
        pltpu.VMEM((C, DP), jnp.float32),
        pltpu.VMEM((C, DP), jnp.float32),
        pltpu.SemaphoreType.DMA,
        pltpu.SemaphoreType.DMA,
        pltpu.SemaphoreType.DMA,
        pltpu.SemaphoreType.DMA,
    ],
)
def _gather_kernel(idx_hbm, table_hbm, out_hbm,
                   idx_v, rows0, rows1, g0, g1, o0, o1):
    wid = lax.axis_index("s") * NC + lax.axis_index("c")
    base = wid * BPW
    pltpu.sync_copy(idx_hbm.at[pl.ds(base, BPW)], idx_v)

    def gather(g, rows, sem):
        return pltpu.make_async_copy(
            table_hbm.at[idx_v.at[pl.ds(g * C, C)]], rows, sem)

    def store(g, rows, sem):
        return pltpu.make_async_copy(
            rows, out_hbm.at[pl.ds(base + g * C, C)], sem)

    gather(0, rows0, g0).start()
    gather(1, rows1, g1).start()

    def pair(i, _):
        a = 2 * i
        gather(a, rows0, g0).wait()
        store(a, rows0, o0).start()
        gather(a + 1, rows1, g1).wait()
        store(a + 1, rows1, o1).start()
        store(a, rows0, o0).wait()
        gather(a + 2, rows0, g0).start()
        store(a + 1, rows1, o1).wait()
        gather(a + 3, rows1, g1).start()
        return 0

    lax.fori_loop(0, NPAIR - 1, pair, 0)

    last = NCHUNK - 2
    gather(last, rows0, g0).wait()
    store(last, rows0, o0).start()
    gather(last + 1, rows1, g1).wait()
    store(last + 1, rows1, o1).start()
    store(last, rows0, o0).wait()
    store(last + 1, rows1, o1).wait()


_VB = 1024
_NVB = (100002 + _VB - 1) // _VB


def _padT_body(tT_ref, out_ref):
    blk = tT_ref[...].T
    out_ref[...] = jnp.concatenate(
        [blk, jnp.zeros((_VB, DP - D), jnp.float32)], axis=1)


_pad_kernel = pl.pallas_call(
    _padT_body,
    grid=(_NVB,),
    in_specs=[pl.BlockSpec((D, _VB), lambda i: (0, i))],
    out_specs=pl.BlockSpec((_VB, DP), lambda i: (i, 0)),
    out_shape=jax.ShapeDtypeStruct((_NVB * _VB, DP), jnp.float32),
)


def kernel(inputs, table):
    idx = inputs.reshape(-1).astype(jnp.int32)
    table_p = _pad_kernel(table.T)
    out = _gather_kernel(idx, table_p)
    return out[:, :D].reshape(inputs.shape[0], inputs.shape[1], D)
